# baseline (speedup 1.0000x reference)
import dataclasses

import jax
import jax.numpy as jnp
from jax import lax
from jax.experimental import pallas as pl
from jax.experimental.pallas import tpu as pltpu
from jax.experimental.pallas import tpu_sc as plsc

_INV_CELL = 64.0
_C = 1600
_L = 16


def _block_body(pos_vmem, sf_vmem, gf_vmem):
    iota = lax.iota(jnp.int32, _L)

    @pl.loop(0, _C, step=_L)
    def _(p0):
        rows = p0 + iota
        rows3 = rows * 3
        rows8 = rows * 8
        rows24 = rows * 24
        px = plsc.load_gather(pos_vmem, [rows3])
        py = plsc.load_gather(pos_vmem, [rows3 + 1])
        pz = plsc.load_gather(pos_vmem, [rows3 + 2])

        rx = px * _INV_CELL
        ry = py * _INV_CELL
        rz = pz * _INV_CELL
        fx = rx - rx.astype(jnp.int32).astype(jnp.float32)
        fy = ry - ry.astype(jnp.int32).astype(jnp.float32)
        fz = rz - rz.astype(jnp.int32).astype(jnp.float32)
        ox = 1.0 - fx
        oy = 1.0 - fy
        oz = 1.0 - fz
        dxp = jnp.sign(fx) * _INV_CELL
        dyp = jnp.sign(fy) * _INV_CELL
        dzp = jnp.sign(fz) * _INV_CELL

        bx = (ox, fx)
        by = (oy, fy)
        bz = (oz, fz)
        dbx = (-dxp, dxp)
        dby = (-dyp, dyp)
        dbz = (-dzp, dzp)

        bxy = {(i, j): bx[i] * by[j] for i in (0, 1) for j in (0, 1)}
        bxz = {(i, k): bx[i] * bz[k] for i in (0, 1) for k in (0, 1)}
        byz = {(j, k): by[j] * bz[k] for j in (0, 1) for k in (0, 1)}

        for w in range(8):
            i, j, k = (w >> 2) & 1, (w >> 1) & 1, w & 1
            plsc.store_scatter(sf_vmem, [rows8 + w], bxy[(i, j)] * bz[k])
            plsc.store_scatter(gf_vmem, [rows24 + 3 * w], dbx[i] * byz[(j, k)])
            plsc.store_scatter(gf_vmem, [rows24 + 3 * w + 1], dby[j] * bxz[(i, k)])
            plsc.store_scatter(gf_vmem, [rows24 + 3 * w + 2], dbz[k] * bxy[(i, j)])


def kernel(position_stack):
    n = position_stack.shape[0]
    mesh = plsc.VectorSubcoreMesh(core_axis_name="core", subcore_axis_name="subcore")
    cp = pltpu.CompilerParams()
    if "needs_layout_passes" in pltpu.CompilerParams.__dataclass_fields__:
        cp = dataclasses.replace(cp, needs_layout_passes=False)

    @pl.kernel(
        out_type=[
            jax.ShapeDtypeStruct((n * 8,), jnp.float32),
            jax.ShapeDtypeStruct((n * 24,), jnp.float32),
        ],
        mesh=mesh,
        compiler_params=cp,
    )
    def run(pos_hbm, sf_hbm, gf_hbm):
        pltpu.emit_pipeline(
            _block_body,
            grid=(n // _C,),
            in_specs=[pl.BlockSpec((_C * 3,), lambda i: (i,))],
            out_specs=[
                pl.BlockSpec((_C * 8,), lambda i: (i,)),
                pl.BlockSpec((_C * 24,), lambda i: (i,)),
            ],
            core_axis_name=("core", "subcore"),
            dimension_semantics=(pltpu.PARALLEL,),
        )(pos_hbm, sf_hbm, gf_hbm)

    sf, gf = run(position_stack.reshape(n * 3))
    return sf.reshape(n, 8), gf.reshape(n, 8, 3)


# speedup vs baseline: 78.9547x; 78.9547x over previous
import dataclasses

import jax
import jax.numpy as jnp
from jax import lax
from jax.experimental import pallas as pl
from jax.experimental.pallas import tpu as pltpu
from jax.experimental.pallas import tpu_sc as plsc

_INV_CELL = 64.0
_C = 1536
_L = 16
_NMAIN = 999936
_NTAIL = 64


def _compute(px, py, pz):
    rx = px * _INV_CELL
    ry = py * _INV_CELL
    rz = pz * _INV_CELL
    fx = rx - rx.astype(jnp.int32).astype(jnp.float32)
    fy = ry - ry.astype(jnp.int32).astype(jnp.float32)
    fz = rz - rz.astype(jnp.int32).astype(jnp.float32)
    ox = 1.0 - fx
    oy = 1.0 - fy
    oz = 1.0 - fz
    dxp = jnp.sign(fx) * _INV_CELL
    dyp = jnp.sign(fy) * _INV_CELL
    dzp = jnp.sign(fz) * _INV_CELL

    bx = (ox, fx)
    by = (oy, fy)
    bz = (oz, fz)
    dbx = (-dxp, dxp)
    dby = (-dyp, dyp)
    dbz = (-dzp, dzp)

    bxy = {(i, j): bx[i] * by[j] for i in (0, 1) for j in (0, 1)}
    bxz = {(i, k): bx[i] * bz[k] for i in (0, 1) for k in (0, 1)}
    byz = {(j, k): by[j] * bz[k] for j in (0, 1) for k in (0, 1)}

    sf = []
    gf = [None] * 24
    for w in range(8):
        i, j, k = (w >> 2) & 1, (w >> 1) & 1, w & 1
        sf.append(bxy[(i, j)] * bz[k])
        gf[w] = dbx[i] * byz[(j, k)]
        gf[8 + w] = dby[j] * bxz[(i, k)]
        gf[16 + w] = dbz[k] * bxy[(i, j)]
    return sf, gf


def _block_body(pos_vmem, sf_vmem, gf_vmem):
    @pl.loop(0, _C, step=_L)
    def _(g):
        s = pl.ds(g, _L)
        sf, gf = _compute(pos_vmem[0, s], pos_vmem[1, s], pos_vmem[2, s])
        for w in range(8):
            sf_vmem[w, s] = sf[w]
        for r in range(24):
            gf_vmem[r, s] = gf[r]


def kernel(position_stack):
    n = position_stack.shape[0]
    assert n == _NMAIN + _NTAIL
    mesh = plsc.VectorSubcoreMesh(core_axis_name="core", subcore_axis_name="subcore")
    cp = pltpu.CompilerParams()
    if "needs_layout_passes" in pltpu.CompilerParams.__dataclass_fields__:
        cp = dataclasses.replace(cp, needs_layout_passes=False)

    @pl.kernel(
        out_type=[
            jax.ShapeDtypeStruct((8, n), jnp.float32),
            jax.ShapeDtypeStruct((24, n), jnp.float32),
        ],
        mesh=mesh,
        compiler_params=cp,
        scratch_types=[
            pltpu.VMEM((3, _NTAIL), jnp.float32),
            pltpu.VMEM((8, _NTAIL), jnp.float32),
            pltpu.VMEM((24, _NTAIL), jnp.float32),
        ],
    )
    def run(pos_hbm, sf_hbm, gf_hbm, tp_v, ts_v, tg_v):
        pltpu.emit_pipeline(
            _block_body,
            grid=(_NMAIN // _C,),
            in_specs=[pl.BlockSpec((3, _C), lambda i: (0, i))],
            out_specs=[
                pl.BlockSpec((8, _C), lambda i: (0, i)),
                pl.BlockSpec((24, _C), lambda i: (0, i)),
            ],
            core_axis_name=("core", "subcore"),
            dimension_semantics=(pltpu.PARALLEL,),
        )(pos_hbm, sf_hbm, gf_hbm)

        wid = lax.axis_index("subcore") * 2 + lax.axis_index("core")

        @pl.when(wid == 0)
        def _():
            pltpu.sync_copy(pos_hbm.at[:, pl.ds(_NMAIN, _NTAIL)], tp_v)

            @pl.loop(0, _NTAIL, step=_L)
            def _(g):
                s = pl.ds(g, _L)
                sf, gf = _compute(tp_v[0, s], tp_v[1, s], tp_v[2, s])
                for w in range(8):
                    ts_v[w, s] = sf[w]
                for r in range(24):
                    tg_v[r, s] = gf[r]

            pltpu.sync_copy(ts_v, sf_hbm.at[:, pl.ds(_NMAIN, _NTAIL)])
            pltpu.sync_copy(tg_v, gf_hbm.at[:, pl.ds(_NMAIN, _NTAIL)])

    sf, gf = run(position_stack.T)
    return sf.T, gf.reshape(3, 8, n).transpose(2, 1, 0)
